# Initial kernel scaffold; baseline (speedup 1.0000x reference)
import jax
import jax.numpy as jnp
from jax.experimental import pallas as pl


def kernel(h, x, edge_index, edge_attr, params):
    raise NotImplementedError("write your pallas kernel here")


# speedup vs baseline: 2.0911x; 2.0911x over previous
import functools

import jax
import jax.numpy as jnp
from jax import lax
from jax.experimental import pallas as pl
from jax.experimental.pallas import tpu as pltpu
from jax.experimental.pallas import tpu_sc as plsc

N = 10000
E = 320000
H = 128
NORM_FACTOR = 100.0

NC = 2
NS = 16
NW = NC * NS
PER_W = E // NW
CH = 80
NCH = PER_W // CH
NPAD = 10240
ROWS_PER_TILE = NPAD // NS

BE = 512
BN = 1000


def _mesh():
    return plsc.VectorSubcoreMesh(
        core_axis_name="c", subcore_axis_name="s", num_cores=NC, num_subcores=NS
    )


def _silu(v):
    return v * (1.0 / (1.0 + jnp.exp(-v)))


def _sc_gather_add(g1, g2, row, col):
    @functools.partial(
        pl.kernel,
        out_type=jax.ShapeDtypeStruct((E, H), jnp.float32),
        mesh=_mesh(),
        scratch_types=[
            pltpu.VMEM((CH,), jnp.int32),
            pltpu.VMEM((CH,), jnp.int32),
            pltpu.VMEM((CH, H), jnp.float32),
            pltpu.VMEM((CH, H), jnp.float32),
            pltpu.SemaphoreType.DMA,
            pltpu.SemaphoreType.DMA,
        ],
    )
    def k(g1_hbm, g2_hbm, row_hbm, col_hbm, out_hbm, idx1, idx2, buf1, buf2, sem1, sem2):
        wid = lax.axis_index("s") * NC + lax.axis_index("c")
        base = wid * PER_W

        def body(i, _):
            off = base + i * CH
            pltpu.sync_copy(row_hbm.at[pl.ds(off, CH)], idx1)
            pltpu.sync_copy(col_hbm.at[pl.ds(off, CH)], idx2)
            c1 = pltpu.async_copy(g1_hbm.at[idx1], buf1, sem1)
            c2 = pltpu.async_copy(g2_hbm.at[idx2], buf2, sem2)
            c1.wait()
            c2.wait()

            def addrow(r, _):
                for l in range(H // 16):
                    sl = pl.ds(l * 16, 16)
                    buf1[r, sl] = buf1[r, sl] + buf2[r, sl]
                return 0

            lax.fori_loop(0, CH, addrow, 0, unroll=False)
            pltpu.sync_copy(buf1, out_hbm.at[pl.ds(off, CH)])
            return 0

        lax.fori_loop(0, NCH, body, 0, unroll=False)

    return k(g1, g2, row, col)


def _sc_coord(xs, ys, zs, row, col):
    @functools.partial(
        pl.kernel,
        out_type=tuple(jax.ShapeDtypeStruct((E,), jnp.float32) for _ in range(4)),
        mesh=_mesh(),
        scratch_types=[
            pltpu.VMEM((N,), jnp.float32),
            pltpu.VMEM((N,), jnp.float32),
            pltpu.VMEM((N,), jnp.float32),
            pltpu.VMEM((CH,), jnp.int32),
            pltpu.VMEM((CH,), jnp.int32),
            pltpu.VMEM((CH,), jnp.float32),
            pltpu.VMEM((CH,), jnp.float32),
            pltpu.VMEM((CH,), jnp.float32),
            pltpu.VMEM((CH,), jnp.float32),
        ],
        compiler_params=pltpu.CompilerParams(needs_layout_passes=False),
    )
    def k(xs_hbm, ys_hbm, zs_hbm, row_hbm, col_hbm,
          dx_hbm, dy_hbm, dz_hbm, rad_hbm,
          xt, yt, zt, idx1, idx2, dxb, dyb, dzb, radb):
        wid = lax.axis_index("s") * NC + lax.axis_index("c")
        base = wid * PER_W
        pltpu.sync_copy(xs_hbm, xt)
        pltpu.sync_copy(ys_hbm, yt)
        pltpu.sync_copy(zs_hbm, zt)

        def body(i, _):
            off = base + i * CH
            pltpu.sync_copy(row_hbm.at[pl.ds(off, CH)], idx1)
            pltpu.sync_copy(col_hbm.at[pl.ds(off, CH)], idx2)

            def group(g, _):
                sl = pl.ds(g * 16, 16)
                r = idx1[sl]
                c = idx2[sl]
                dx = plsc.load_gather(xt, [r]) - plsc.load_gather(xt, [c])
                dy = plsc.load_gather(yt, [r]) - plsc.load_gather(yt, [c])
                dz = plsc.load_gather(zt, [r]) - plsc.load_gather(zt, [c])
                dxb[sl] = dx
                dyb[sl] = dy
                dzb[sl] = dz
                radb[sl] = dx * dx + dy * dy + dz * dz
                return 0

            lax.fori_loop(0, CH // 16, group, 0, unroll=False)
            pltpu.sync_copy(dxb, dx_hbm.at[pl.ds(off, CH)])
            pltpu.sync_copy(dyb, dy_hbm.at[pl.ds(off, CH)])
            pltpu.sync_copy(dzb, dz_hbm.at[pl.ds(off, CH)])
            pltpu.sync_copy(radb, rad_hbm.at[pl.ds(off, CH)])
            return 0

        lax.fori_loop(0, NCH, body, 0, unroll=False)

    return k(xs, ys, zs, row, col)


def _sc_scatter_add(vals, row, zeros):
    @functools.partial(
        pl.kernel,
        out_type=jax.ShapeDtypeStruct((NC, NPAD, H), jnp.float32),
        mesh=_mesh(),
        scratch_types=[
            pltpu.VMEM((CH,), jnp.int32),
            pltpu.VMEM((CH, H), jnp.float32),
            pltpu.VMEM_SHARED((NPAD, H), jnp.float32),
            pltpu.SemaphoreType.DMA,
        ],
    )
    def k(vals_hbm, row_hbm, zero_hbm, out_hbm, idx, buf, acc, sem):
        c = lax.axis_index("c")
        s = lax.axis_index("s")
        wid = s * NC + c
        r0 = s * ROWS_PER_TILE
        pltpu.sync_copy(
            zero_hbm.at[pl.ds(r0, ROWS_PER_TILE)], acc.at[pl.ds(r0, ROWS_PER_TILE)]
        )
        plsc.subcore_barrier()

        base = wid * PER_W

        def body(i, _):
            off = base + i * CH
            pltpu.sync_copy(row_hbm.at[pl.ds(off, CH)], idx)
            pltpu.sync_copy(vals_hbm.at[pl.ds(off, CH)], buf)
            pltpu.sync_copy(buf, acc.at[idx], add=True)
            return 0

        lax.fori_loop(0, NCH, body, 0, unroll=False)
        plsc.subcore_barrier()
        pltpu.sync_copy(
            acc.at[pl.ds(r0, ROWS_PER_TILE)], out_hbm.at[c, pl.ds(r0, ROWS_PER_TILE)]
        )

    return k(vals, row, zeros)


def _tc_project(hh, wcat):
    def body(h_ref, w_ref, g1_ref, g2_ref):
        g = jnp.dot(h_ref[...], w_ref[...], preferred_element_type=jnp.float32)
        g1_ref[...] = g[:, :H]
        g2_ref[...] = g[:, H:]

    return pl.pallas_call(
        body,
        grid=(N // BN,),
        in_specs=[
            pl.BlockSpec((BN, H), lambda i: (i, 0)),
            pl.BlockSpec((H, 2 * H), lambda i: (0, 0)),
        ],
        out_specs=[pl.BlockSpec((BN, H), lambda i: (i, 0))] * 2,
        out_shape=[jax.ShapeDtypeStruct((N, H), jnp.float32)] * 2,
    )(hh, wcat)


def _tc_edge_mlp(s, rad, ea, w2, smalls):
    def body(s_ref, rad_ref, ea_ref, w2_ref, sm_ref, out_ref):
        m1 = (
            s_ref[...]
            + sm_ref[0:1, :]
            + rad_ref[...] * sm_ref[1:2, :]
            + ea_ref[...] * sm_ref[2:3, :]
        )
        m1 = _silu(m1)
        m = jnp.dot(m1, w2_ref[...], preferred_element_type=jnp.float32) + sm_ref[3:4, :]
        m = _silu(m)
        att = jnp.sum(m * sm_ref[4:5, :], axis=1, keepdims=True) + sm_ref[5:6, 0:1]
        out_ref[...] = m * (1.0 / (1.0 + jnp.exp(-att)))

    return pl.pallas_call(
        body,
        grid=(E // BE,),
        in_specs=[
            pl.BlockSpec((BE, H), lambda i: (i, 0)),
            pl.BlockSpec((BE, 1), lambda i: (i, 0)),
            pl.BlockSpec((BE, 1), lambda i: (i, 0)),
            pl.BlockSpec((H, H), lambda i: (0, 0)),
            pl.BlockSpec((8, H), lambda i: (0, 0)),
        ],
        out_specs=pl.BlockSpec((BE, H), lambda i: (i, 0)),
        out_shape=jax.ShapeDtypeStruct((E, H), jnp.float32),
    )(s, rad, ea, w2, smalls)


def _tc_edge_equiv(s, rad, ea, dx, dy, dz, w2, smalls):
    def body(s_ref, rad_ref, ea_ref, dx_ref, dy_ref, dz_ref, w2_ref, sm_ref, out_ref):
        radial = rad_ref[...]
        m1 = (
            s_ref[...]
            + sm_ref[0:1, :]
            + radial * sm_ref[1:2, :]
            + ea_ref[...] * sm_ref[2:3, :]
        )
        m1 = _silu(m1)
        cfeat = jnp.dot(m1, w2_ref[...], preferred_element_type=jnp.float32) + sm_ref[3:4, :]
        cfeat = _silu(cfeat)
        phi = jnp.sum(cfeat * sm_ref[4:5, :], axis=1, keepdims=True)
        scale = phi / (jnp.sqrt(radial + 1e-8) + 1.0)
        lane = lax.broadcasted_iota(jnp.int32, (1, H), 1)
        trans = scale * (
            dx_ref[...] * (lane == 0).astype(jnp.float32)
            + dy_ref[...] * (lane == 1).astype(jnp.float32)
            + dz_ref[...] * (lane == 2).astype(jnp.float32)
        )
        out_ref[...] = trans

    return pl.pallas_call(
        body,
        grid=(E // BE,),
        in_specs=[
            pl.BlockSpec((BE, H), lambda i: (i, 0)),
            pl.BlockSpec((BE, 1), lambda i: (i, 0)),
            pl.BlockSpec((BE, 1), lambda i: (i, 0)),
            pl.BlockSpec((BE, 1), lambda i: (i, 0)),
            pl.BlockSpec((BE, 1), lambda i: (i, 0)),
            pl.BlockSpec((BE, 1), lambda i: (i, 0)),
            pl.BlockSpec((H, H), lambda i: (0, 0)),
            pl.BlockSpec((8, H), lambda i: (0, 0)),
        ],
        out_specs=pl.BlockSpec((BE, H), lambda i: (i, 0)),
        out_shape=jax.ShapeDtypeStruct((E, H), jnp.float32),
    )(s, rad, ea, dx, dy, dz, w2, smalls)


def _tc_node(hh, parts, n1, n2, smalls):
    def body(h_ref, pa_ref, pb_ref, n1_ref, n2_ref, sm_ref, out_ref):
        hcur = h_ref[...]
        agg = (pa_ref[0] + pb_ref[0]) * (1.0 / NORM_FACTOR)
        u = (
            jnp.dot(hcur, n1_ref[:H, :], preferred_element_type=jnp.float32)
            + jnp.dot(agg, n1_ref[H:, :], preferred_element_type=jnp.float32)
            + sm_ref[0:1, :]
        )
        u = _silu(u)
        u = jnp.dot(u, n2_ref[...], preferred_element_type=jnp.float32) + sm_ref[1:2, :]
        out_ref[...] = hcur + u

    return pl.pallas_call(
        body,
        grid=(N // BN,),
        in_specs=[
            pl.BlockSpec((BN, H), lambda i: (i, 0)),
            pl.BlockSpec((1, BN, H), lambda i: (0, i, 0)),
            pl.BlockSpec((1, BN, H), lambda i: (1, i, 0)),
            pl.BlockSpec((2 * H, H), lambda i: (0, 0)),
            pl.BlockSpec((H, H), lambda i: (0, 0)),
            pl.BlockSpec((8, H), lambda i: (0, 0)),
        ],
        out_specs=pl.BlockSpec((BN, H), lambda i: (i, 0)),
        out_shape=jax.ShapeDtypeStruct((N, H), jnp.float32),
    )(hh, parts, parts, n1, n2, smalls)


def _tc_final_x(xp, parts):
    def body(x_ref, pa_ref, pb_ref, out_ref):
        out_ref[...] = x_ref[...] + (pa_ref[0] + pb_ref[0]) * (1.0 / NORM_FACTOR)

    return pl.pallas_call(
        body,
        grid=(N // BN,),
        in_specs=[
            pl.BlockSpec((BN, H), lambda i: (i, 0)),
            pl.BlockSpec((1, BN, H), lambda i: (0, i, 0)),
            pl.BlockSpec((1, BN, H), lambda i: (1, i, 0)),
        ],
        out_specs=pl.BlockSpec((BN, H), lambda i: (i, 0)),
        out_shape=jax.ShapeDtypeStruct((N, H), jnp.float32),
    )(xp, parts, parts)


def _pack_gcl_smalls(p):
    z = jnp.zeros((8, H), jnp.float32)
    z = z.at[0].set(p["e1"]["b"])
    z = z.at[1].set(p["e1"]["w"][2 * H])
    z = z.at[2].set(p["e1"]["w"][2 * H + 1])
    z = z.at[3].set(p["e2"]["b"])
    z = z.at[4].set(p["att"]["w"][:, 0])
    z = z.at[5].set(jnp.full((H,), p["att"]["b"][0]))
    return z


def _pack_equiv_smalls(p):
    z = jnp.zeros((8, H), jnp.float32)
    z = z.at[0].set(p["c1"]["b"])
    z = z.at[1].set(p["c1"]["w"][2 * H])
    z = z.at[2].set(p["c1"]["w"][2 * H + 1])
    z = z.at[3].set(p["c2"]["b"])
    z = z.at[4].set(p["c3w"][:, 0])
    return z


def kernel(h, x, edge_index, edge_attr, params):
    row = edge_index[0]
    col = edge_index[1]
    zeros_h = jnp.zeros((NPAD, H), jnp.float32)

    dx, dy, dz, rad = _sc_coord(
        jnp.asarray(x[:, 0]), jnp.asarray(x[:, 1]), jnp.asarray(x[:, 2]), row, col
    )
    rad1 = rad.reshape(E, 1)

    hh = h
    for i in range(2):
        p = params["gcl%d" % i]
        w1 = p["e1"]["w"]
        wcat = jnp.concatenate([w1[:H], w1[H : 2 * H]], axis=1)
        g1, g2 = _tc_project(hh, wcat)
        s = _sc_gather_add(g1, g2, row, col)
        ef = _tc_edge_mlp(s, rad1, edge_attr, p["e2"]["w"], _pack_gcl_smalls(p))
        parts = _sc_scatter_add(ef, row, zeros_h)
        hh = _tc_node(hh, parts, p["n1"]["w"], p["n2"]["w"],
                      jnp.stack([p["n1"]["b"], p["n2"]["b"]] + [jnp.zeros((H,))] * 6))

    pe = params["equiv"]
    c1 = pe["c1"]["w"]
    wcat = jnp.concatenate([c1[:H], c1[H : 2 * H]], axis=1)
    g1, g2 = _tc_project(hh, wcat)
    s = _sc_gather_add(g1, g2, row, col)
    tr = _tc_edge_equiv(s, rad1, edge_attr,
                        dx.reshape(E, 1), dy.reshape(E, 1), dz.reshape(E, 1),
                        pe["c2"]["w"], _pack_equiv_smalls(pe))
    parts = _sc_scatter_add(tr, row, zeros_h)
    xp = jnp.concatenate([x, jnp.zeros((N, H - 3), jnp.float32)], axis=1)
    x16 = _tc_final_x(xp, parts)
    xx = x16[:, :3]
    return hh, xx


# speedup vs baseline: 2.5213x; 1.2057x over previous
import functools

import jax
import jax.numpy as jnp
from jax import lax
from jax.experimental import pallas as pl
from jax.experimental.pallas import tpu as pltpu
from jax.experimental.pallas import tpu_sc as plsc

N = 10000
E = 320000
H = 128
NORM_FACTOR = 100.0

NC = 2
NS = 16
NW = NC * NS
PER_W = E // NW
CH = 80
NCH = PER_W // CH
NB = 3
NPAD = 10240
ROWS_PER_TILE = NPAD // NS

BE = 512
BN = 1000


def _mesh():
    return plsc.VectorSubcoreMesh(
        core_axis_name="c", subcore_axis_name="s", num_cores=NC, num_subcores=NS
    )


def _silu(v):
    return v * (1.0 / (1.0 + jnp.exp(-v)))


def _sc_gather_add(g1, g2, row2d, col2d):
    @functools.partial(
        pl.kernel,
        out_type=jax.ShapeDtypeStruct((E, H), jnp.float32),
        mesh=_mesh(),
        scratch_types=(
            [pltpu.VMEM((NCH, CH), jnp.int32)] * 2
            + [pltpu.VMEM((CH, H), jnp.float32)] * (3 * NB)
            + [pltpu.SemaphoreType.DMA] * (3 * NB)
        ),
    )
    def k(g1_hbm, g2_hbm, row_hbm, col_hbm, out_hbm, idxr, idxc, *rest):
        bufs1 = rest[0:NB]
        bufs2 = rest[NB : 2 * NB]
        obufs = rest[2 * NB : 3 * NB]
        sem1 = rest[3 * NB : 4 * NB]
        sem2 = rest[4 * NB : 5 * NB]
        wsem = rest[5 * NB : 6 * NB]
        wid = lax.axis_index("s") * NC + lax.axis_index("c")
        base = wid * PER_W
        pltpu.sync_copy(row_hbm.at[wid], idxr)
        pltpu.sync_copy(col_hbm.at[wid], idxc)
        for b in range(NB):
            pltpu.async_copy(g1_hbm.at[idxr.at[b]], bufs1[b], sem1[b])
            pltpu.async_copy(g2_hbm.at[idxc.at[b]], bufs2[b], sem2[b])

        nrounds = (NCH + NB - 1) // NB

        def body(j, _):
            for b in range(NB):
                i = j * NB + b

                @pl.when(i < NCH)
                def _():
                    pltpu.make_async_copy(g1_hbm.at[idxr.at[i]], bufs1[b], sem1[b]).wait()
                    pltpu.make_async_copy(g2_hbm.at[idxc.at[i]], bufs2[b], sem2[b]).wait()

                    @pl.when(i >= NB)
                    def _():
                        pltpu.make_async_copy(
                            obufs[b], out_hbm.at[pl.ds(base, CH)], wsem[b]
                        ).wait()

                    def addrow(r, _):
                        for l in range(H // 16):
                            sl = pl.ds(l * 16, 16)
                            obufs[b][r, sl] = bufs1[b][r, sl] + bufs2[b][r, sl]
                        return 0

                    lax.fori_loop(0, CH, addrow, 0, unroll=2)

                    @pl.when(i + NB < NCH)
                    def _():
                        pltpu.async_copy(g1_hbm.at[idxr.at[i + NB]], bufs1[b], sem1[b])
                        pltpu.async_copy(g2_hbm.at[idxc.at[i + NB]], bufs2[b], sem2[b])

                    pltpu.async_copy(
                        obufs[b], out_hbm.at[pl.ds(base + i * CH, CH)], wsem[b]
                    )
            return 0

        lax.fori_loop(0, nrounds, body, 0, unroll=False)
        for b in range(NB):
            pltpu.make_async_copy(obufs[b], out_hbm.at[pl.ds(base, CH)], wsem[b]).wait()

    return k(g1, g2, row2d, col2d)


def _sc_coord(xs, ys, zs, row, col):
    @functools.partial(
        pl.kernel,
        out_type=tuple(jax.ShapeDtypeStruct((E,), jnp.float32) for _ in range(4)),
        mesh=_mesh(),
        scratch_types=(
            [pltpu.VMEM((N,), jnp.float32)] * 3
            + [pltpu.VMEM((PER_W,), jnp.int32)] * 2
            + [pltpu.VMEM((PER_W,), jnp.float32)] * 4
        ),
        compiler_params=pltpu.CompilerParams(needs_layout_passes=False),
    )
    def k(xs_hbm, ys_hbm, zs_hbm, row_hbm, col_hbm,
          dx_hbm, dy_hbm, dz_hbm, rad_hbm,
          xt, yt, zt, idx1, idx2, dxb, dyb, dzb, radb):
        wid = lax.axis_index("s") * NC + lax.axis_index("c")
        base = wid * PER_W
        pltpu.sync_copy(xs_hbm, xt)
        pltpu.sync_copy(ys_hbm, yt)
        pltpu.sync_copy(zs_hbm, zt)
        pltpu.sync_copy(row_hbm.at[pl.ds(base, PER_W)], idx1)
        pltpu.sync_copy(col_hbm.at[pl.ds(base, PER_W)], idx2)

        def group(g, _):
            sl = pl.ds(g * 16, 16)
            r = idx1[sl]
            c = idx2[sl]
            dx = plsc.load_gather(xt, [r]) - plsc.load_gather(xt, [c])
            dy = plsc.load_gather(yt, [r]) - plsc.load_gather(yt, [c])
            dz = plsc.load_gather(zt, [r]) - plsc.load_gather(zt, [c])
            dxb[sl] = dx
            dyb[sl] = dy
            dzb[sl] = dz
            radb[sl] = dx * dx + dy * dy + dz * dz
            return 0

        lax.fori_loop(0, PER_W // 16, group, 0, unroll=2)
        pltpu.sync_copy(dxb, dx_hbm.at[pl.ds(base, PER_W)])
        pltpu.sync_copy(dyb, dy_hbm.at[pl.ds(base, PER_W)])
        pltpu.sync_copy(dzb, dz_hbm.at[pl.ds(base, PER_W)])
        pltpu.sync_copy(radb, rad_hbm.at[pl.ds(base, PER_W)])

    return k(xs, ys, zs, row, col)


def _sc_scatter_add(vals, row2d, zeros):
    @functools.partial(
        pl.kernel,
        out_type=jax.ShapeDtypeStruct((NC, NPAD, H), jnp.float32),
        mesh=_mesh(),
        scratch_types=(
            [pltpu.VMEM((NCH, CH), jnp.int32)]
            + [pltpu.VMEM((CH, H), jnp.float32)] * NB
            + [pltpu.VMEM_SHARED((NPAD, H), jnp.float32)]
            + [pltpu.SemaphoreType.DMA] * NB
        ),
    )
    def k(vals_hbm, row_hbm, zero_hbm, out_hbm, idx2d, *rest):
        bufs = rest[0:NB]
        acc = rest[NB]
        sems = rest[NB + 1 : 2 * NB + 1]
        c = lax.axis_index("c")
        s = lax.axis_index("s")
        wid = s * NC + c
        r0 = s * ROWS_PER_TILE
        base = wid * PER_W
        pltpu.sync_copy(
            zero_hbm.at[pl.ds(r0, ROWS_PER_TILE)], acc.at[pl.ds(r0, ROWS_PER_TILE)]
        )
        pltpu.sync_copy(row_hbm.at[wid], idx2d)
        for b in range(NB):
            pltpu.async_copy(vals_hbm.at[pl.ds(base + b * CH, CH)], bufs[b], sems[b])
        plsc.subcore_barrier()

        nrounds = (NCH + NB - 1) // NB

        def body(j, _):
            for b in range(NB):
                i = j * NB + b

                @pl.when(i < NCH)
                def _():
                    pltpu.make_async_copy(
                        vals_hbm.at[pl.ds(base, CH)], bufs[b], sems[b]
                    ).wait()
                    pltpu.sync_copy(bufs[b], acc.at[idx2d.at[i]], add=True)

                    @pl.when(i + NB < NCH)
                    def _():
                        pltpu.async_copy(
                            vals_hbm.at[pl.ds(base + (i + NB) * CH, CH)], bufs[b], sems[b]
                        )
            return 0

        lax.fori_loop(0, nrounds, body, 0, unroll=False)
        plsc.subcore_barrier()
        pltpu.sync_copy(
            acc.at[pl.ds(r0, ROWS_PER_TILE)], out_hbm.at[c, pl.ds(r0, ROWS_PER_TILE)]
        )

    return k(vals, row2d, zeros)


def _tc_project(hh, wcat):
    def body(h_ref, w_ref, g1_ref, g2_ref):
        g = jnp.dot(h_ref[...], w_ref[...], preferred_element_type=jnp.float32)
        g1_ref[...] = g[:, :H]
        g2_ref[...] = g[:, H:]

    return pl.pallas_call(
        body,
        grid=(N // BN,),
        in_specs=[
            pl.BlockSpec((BN, H), lambda i: (i, 0)),
            pl.BlockSpec((H, 2 * H), lambda i: (0, 0)),
        ],
        out_specs=[pl.BlockSpec((BN, H), lambda i: (i, 0))] * 2,
        out_shape=[jax.ShapeDtypeStruct((N, H), jnp.float32)] * 2,
    )(hh, wcat)


def _tc_edge_mlp(s, rad, ea, w2, smalls):
    def body(s_ref, rad_ref, ea_ref, w2_ref, sm_ref, out_ref):
        m1 = (
            s_ref[...]
            + sm_ref[0:1, :]
            + rad_ref[...] * sm_ref[1:2, :]
            + ea_ref[...] * sm_ref[2:3, :]
        )
        m1 = _silu(m1)
        m = jnp.dot(m1, w2_ref[...], preferred_element_type=jnp.float32) + sm_ref[3:4, :]
        m = _silu(m)
        att = jnp.sum(m * sm_ref[4:5, :], axis=1, keepdims=True) + sm_ref[5:6, 0:1]
        out_ref[...] = m * (1.0 / (1.0 + jnp.exp(-att)))

    return pl.pallas_call(
        body,
        grid=(E // BE,),
        in_specs=[
            pl.BlockSpec((BE, H), lambda i: (i, 0)),
            pl.BlockSpec((BE, 1), lambda i: (i, 0)),
            pl.BlockSpec((BE, 1), lambda i: (i, 0)),
            pl.BlockSpec((H, H), lambda i: (0, 0)),
            pl.BlockSpec((8, H), lambda i: (0, 0)),
        ],
        out_specs=pl.BlockSpec((BE, H), lambda i: (i, 0)),
        out_shape=jax.ShapeDtypeStruct((E, H), jnp.float32),
    )(s, rad, ea, w2, smalls)


def _tc_edge_equiv(s, rad, ea, dx, dy, dz, w2, smalls):
    def body(s_ref, rad_ref, ea_ref, dx_ref, dy_ref, dz_ref, w2_ref, sm_ref, out_ref):
        radial = rad_ref[...]
        m1 = (
            s_ref[...]
            + sm_ref[0:1, :]
            + radial * sm_ref[1:2, :]
            + ea_ref[...] * sm_ref[2:3, :]
        )
        m1 = _silu(m1)
        cfeat = jnp.dot(m1, w2_ref[...], preferred_element_type=jnp.float32) + sm_ref[3:4, :]
        cfeat = _silu(cfeat)
        phi = jnp.sum(cfeat * sm_ref[4:5, :], axis=1, keepdims=True)
        scale = phi / (jnp.sqrt(radial + 1e-8) + 1.0)
        lane = lax.broadcasted_iota(jnp.int32, (1, H), 1)
        trans = scale * (
            dx_ref[...] * (lane == 0).astype(jnp.float32)
            + dy_ref[...] * (lane == 1).astype(jnp.float32)
            + dz_ref[...] * (lane == 2).astype(jnp.float32)
        )
        out_ref[...] = trans

    return pl.pallas_call(
        body,
        grid=(E // BE,),
        in_specs=[
            pl.BlockSpec((BE, H), lambda i: (i, 0)),
            pl.BlockSpec((BE, 1), lambda i: (i, 0)),
            pl.BlockSpec((BE, 1), lambda i: (i, 0)),
            pl.BlockSpec((BE, 1), lambda i: (i, 0)),
            pl.BlockSpec((BE, 1), lambda i: (i, 0)),
            pl.BlockSpec((BE, 1), lambda i: (i, 0)),
            pl.BlockSpec((H, H), lambda i: (0, 0)),
            pl.BlockSpec((8, H), lambda i: (0, 0)),
        ],
        out_specs=pl.BlockSpec((BE, H), lambda i: (i, 0)),
        out_shape=jax.ShapeDtypeStruct((E, H), jnp.float32),
    )(s, rad, ea, dx, dy, dz, w2, smalls)


def _tc_node(hh, parts, n1, n2, smalls):
    def body(h_ref, pa_ref, pb_ref, n1_ref, n2_ref, sm_ref, out_ref):
        hcur = h_ref[...]
        agg = (pa_ref[0] + pb_ref[0]) * (1.0 / NORM_FACTOR)
        u = (
            jnp.dot(hcur, n1_ref[:H, :], preferred_element_type=jnp.float32)
            + jnp.dot(agg, n1_ref[H:, :], preferred_element_type=jnp.float32)
            + sm_ref[0:1, :]
        )
        u = _silu(u)
        u = jnp.dot(u, n2_ref[...], preferred_element_type=jnp.float32) + sm_ref[1:2, :]
        out_ref[...] = hcur + u

    return pl.pallas_call(
        body,
        grid=(N // BN,),
        in_specs=[
            pl.BlockSpec((BN, H), lambda i: (i, 0)),
            pl.BlockSpec((1, BN, H), lambda i: (0, i, 0)),
            pl.BlockSpec((1, BN, H), lambda i: (1, i, 0)),
            pl.BlockSpec((2 * H, H), lambda i: (0, 0)),
            pl.BlockSpec((H, H), lambda i: (0, 0)),
            pl.BlockSpec((8, H), lambda i: (0, 0)),
        ],
        out_specs=pl.BlockSpec((BN, H), lambda i: (i, 0)),
        out_shape=jax.ShapeDtypeStruct((N, H), jnp.float32),
    )(hh, parts, parts, n1, n2, smalls)


def _tc_final_x(xp, parts):
    def body(x_ref, pa_ref, pb_ref, out_ref):
        out_ref[...] = x_ref[...] + (pa_ref[0] + pb_ref[0]) * (1.0 / NORM_FACTOR)

    return pl.pallas_call(
        body,
        grid=(N // BN,),
        in_specs=[
            pl.BlockSpec((BN, H), lambda i: (i, 0)),
            pl.BlockSpec((1, BN, H), lambda i: (0, i, 0)),
            pl.BlockSpec((1, BN, H), lambda i: (1, i, 0)),
        ],
        out_specs=pl.BlockSpec((BN, H), lambda i: (i, 0)),
        out_shape=jax.ShapeDtypeStruct((N, H), jnp.float32),
    )(xp, parts, parts)


def _pack_gcl_smalls(p):
    z = jnp.zeros((8, H), jnp.float32)
    z = z.at[0].set(p["e1"]["b"])
    z = z.at[1].set(p["e1"]["w"][2 * H])
    z = z.at[2].set(p["e1"]["w"][2 * H + 1])
    z = z.at[3].set(p["e2"]["b"])
    z = z.at[4].set(p["att"]["w"][:, 0])
    z = z.at[5].set(jnp.full((H,), p["att"]["b"][0]))
    return z


def _pack_equiv_smalls(p):
    z = jnp.zeros((8, H), jnp.float32)
    z = z.at[0].set(p["c1"]["b"])
    z = z.at[1].set(p["c1"]["w"][2 * H])
    z = z.at[2].set(p["c1"]["w"][2 * H + 1])
    z = z.at[3].set(p["c2"]["b"])
    z = z.at[4].set(p["c3w"][:, 0])
    return z


def kernel(h, x, edge_index, edge_attr, params):
    row = edge_index[0]
    col = edge_index[1]
    row2d = row.reshape(NW, NCH, CH)
    col2d = col.reshape(NW, NCH, CH)
    zeros_h = jnp.zeros((NPAD, H), jnp.float32)

    dx, dy, dz, rad = _sc_coord(
        jnp.asarray(x[:, 0]), jnp.asarray(x[:, 1]), jnp.asarray(x[:, 2]), row, col
    )
    rad1 = rad.reshape(E, 1)

    hh = h
    for i in range(2):
        p = params["gcl%d" % i]
        w1 = p["e1"]["w"]
        wcat = jnp.concatenate([w1[:H], w1[H : 2 * H]], axis=1)
        g1, g2 = _tc_project(hh, wcat)
        s = _sc_gather_add(g1, g2, row2d, col2d)
        ef = _tc_edge_mlp(s, rad1, edge_attr, p["e2"]["w"], _pack_gcl_smalls(p))
        parts = _sc_scatter_add(ef, row2d, zeros_h)
        hh = _tc_node(hh, parts, p["n1"]["w"], p["n2"]["w"],
                      jnp.stack([p["n1"]["b"], p["n2"]["b"]] + [jnp.zeros((H,))] * 6))

    pe = params["equiv"]
    c1 = pe["c1"]["w"]
    wcat = jnp.concatenate([c1[:H], c1[H : 2 * H]], axis=1)
    g1, g2 = _tc_project(hh, wcat)
    s = _sc_gather_add(g1, g2, row2d, col2d)
    tr = _tc_edge_equiv(s, rad1, edge_attr,
                        dx.reshape(E, 1), dy.reshape(E, 1), dz.reshape(E, 1),
                        pe["c2"]["w"], _pack_equiv_smalls(pe))
    parts = _sc_scatter_add(tr, row2d, zeros_h)
    xp = jnp.concatenate([x, jnp.zeros((N, H - 3), jnp.float32)], axis=1)
    x16 = _tc_final_x(xp, parts)
    xx = x16[:, :3]
    return hh, xx


# speedup vs baseline: 3.2100x; 1.2732x over previous
import functools

import jax
import jax.numpy as jnp
from jax import lax
from jax.experimental import pallas as pl
from jax.experimental.pallas import tpu as pltpu
from jax.experimental.pallas import tpu_sc as plsc

N = 10000
E = 320000
H = 128
NORM_FACTOR = 100.0

NC = 2
NS = 16
NW = NC * NS
CH = 128
NCH = 80
PER_W = NCH * CH
EPAD = NW * PER_W
NB = 2
NBS = 2
XW = 16
NPAD = 10240
ROWS_PER_TILE = NPAD // NS

BE = 2048
BN = 1000


def _mesh():
    return plsc.VectorSubcoreMesh(
        core_axis_name="c", subcore_axis_name="s", num_cores=NC, num_subcores=NS
    )


def _silu(v):
    return v * (1.0 / (1.0 + jnp.exp(-v)))


def _rsqrt16(v):
    i = plsc.bitcast(v, jnp.int32)
    i = jnp.int32(0x5F3759DF) - lax.shift_right_arithmetic(i, 1)
    y = plsc.bitcast(i, jnp.float32)
    for _ in range(3):
        y = y * (1.5 - 0.5 * v * y * y)
    return y


def _sc_coord(xs, ys, zs, row, col):
    @functools.partial(
        pl.kernel,
        out_type=tuple(jax.ShapeDtypeStruct((EPAD,), jnp.float32) for _ in range(4)),
        mesh=_mesh(),
        scratch_types=(
            [pltpu.VMEM((N,), jnp.float32)] * 3
            + [pltpu.VMEM((PER_W,), jnp.int32)] * 2
            + [pltpu.VMEM((PER_W,), jnp.float32)] * 4
        ),
        compiler_params=pltpu.CompilerParams(needs_layout_passes=False),
    )
    def k(xs_hbm, ys_hbm, zs_hbm, row_hbm, col_hbm,
          rad_hbm, cdx_hbm, cdy_hbm, cdz_hbm,
          xt, yt, zt, idx1, idx2, radb, cdxb, cdyb, cdzb):
        wid = lax.axis_index("s") * NC + lax.axis_index("c")
        base = wid * PER_W
        pltpu.sync_copy(xs_hbm, xt)
        pltpu.sync_copy(ys_hbm, yt)
        pltpu.sync_copy(zs_hbm, zt)
        pltpu.sync_copy(row_hbm.at[pl.ds(base, PER_W)], idx1)
        pltpu.sync_copy(col_hbm.at[pl.ds(base, PER_W)], idx2)

        def group(g, _):
            sl = pl.ds(g * 16, 16)
            r = idx1[sl]
            c = idx2[sl]
            dx = plsc.load_gather(xt, [r]) - plsc.load_gather(xt, [c])
            dy = plsc.load_gather(yt, [r]) - plsc.load_gather(yt, [c])
            dz = plsc.load_gather(zt, [r]) - plsc.load_gather(zt, [c])
            rad = dx * dx + dy * dy + dz * dz
            y = _rsqrt16(rad + 1e-8)
            s = y / (1.0 + y)
            radb[sl] = rad
            cdxb[sl] = dx * s
            cdyb[sl] = dy * s
            cdzb[sl] = dz * s
            return 0

        lax.fori_loop(0, PER_W // 16, group, 0, unroll=2)
        pltpu.sync_copy(radb, rad_hbm.at[pl.ds(base, PER_W)])
        pltpu.sync_copy(cdxb, cdx_hbm.at[pl.ds(base, PER_W)])
        pltpu.sync_copy(cdyb, cdy_hbm.at[pl.ds(base, PER_W)])
        pltpu.sync_copy(cdzb, cdz_hbm.at[pl.ds(base, PER_W)])

    return k(xs, ys, zs, row, col)


def _sc_gather_add(g1, g2, row3d, col3d, rad, ea, wsm):
    @functools.partial(
        pl.kernel,
        out_type=jax.ShapeDtypeStruct((EPAD, H), jnp.float32),
        mesh=_mesh(),
        scratch_types=(
            [pltpu.VMEM((NCH, CH), jnp.int32)] * 2
            + [pltpu.VMEM((CH, H), jnp.float32)] * (2 * NB)
            + [pltpu.VMEM((CH, H), jnp.float32)]
            + [pltpu.VMEM((PER_W,), jnp.float32)] * 2
            + [pltpu.VMEM((8, H), jnp.float32)]
            + [pltpu.SemaphoreType.DMA] * (2 * NB + 1)
        ),
    )
    def k(g1_hbm, g2_hbm, row_hbm, col_hbm, rad_hbm, ea_hbm, wsm_hbm, out_hbm,
          idxr, idxc, *rest):
        bufs1 = rest[0:NB]
        bufs2 = rest[NB : 2 * NB]
        obuf = rest[2 * NB]
        radw = rest[2 * NB + 1]
        eaw = rest[2 * NB + 2]
        wv = rest[2 * NB + 3]
        sem1 = rest[2 * NB + 4 : 3 * NB + 4]
        sem2 = rest[3 * NB + 4 : 4 * NB + 4]
        wsem = rest[4 * NB + 4]
        wid = lax.axis_index("s") * NC + lax.axis_index("c")
        base = wid * PER_W
        pltpu.sync_copy(row_hbm.at[wid], idxr)
        pltpu.sync_copy(col_hbm.at[wid], idxc)
        pltpu.sync_copy(rad_hbm.at[pl.ds(base, PER_W)], radw)
        pltpu.sync_copy(ea_hbm.at[pl.ds(base, PER_W)], eaw)
        pltpu.sync_copy(wsm_hbm, wv)
        for b in range(NB):
            pltpu.async_copy(g1_hbm.at[idxr.at[b]], bufs1[b], sem1[b])
            pltpu.async_copy(g2_hbm.at[idxc.at[b]], bufs2[b], sem2[b])

        wrv = [wv[0, pl.ds(l * 16, 16)] for l in range(H // 16)]
        wav = [wv[1, pl.ds(l * 16, 16)] for l in range(H // 16)]

        nrounds = NCH // NB

        def body(j, _):
            for b in range(NB):
                i = j * NB + b
                pltpu.make_async_copy(g1_hbm.at[idxr.at[i]], bufs1[b], sem1[b]).wait()
                pltpu.make_async_copy(g2_hbm.at[idxc.at[i]], bufs2[b], sem2[b]).wait()

                @pl.when(i > 0)
                def _():
                    pltpu.make_async_copy(
                        obuf, out_hbm.at[pl.ds(base, CH)], wsem
                    ).wait()

                def addgrp(g, _):
                    rs16 = radw[pl.ds(i * CH + g * 16, 16)]
                    es16 = eaw[pl.ds(i * CH + g * 16, 16)]
                    for rr in range(16):
                        r = g * 16 + rr
                        rs = rs16[rr]
                        es = es16[rr]
                        for l in range(H // 16):
                            sl = pl.ds(l * 16, 16)
                            obuf[r, sl] = (
                                bufs1[b][r, sl] + bufs2[b][r, sl]
                                + rs * wrv[l] + es * wav[l]
                            )
                    return 0

                lax.fori_loop(0, CH // 16, addgrp, 0, unroll=False)

                @pl.when(i + NB < NCH)
                def _():
                    pltpu.async_copy(g1_hbm.at[idxr.at[i + NB]], bufs1[b], sem1[b])
                    pltpu.async_copy(g2_hbm.at[idxc.at[i + NB]], bufs2[b], sem2[b])

                pltpu.async_copy(obuf, out_hbm.at[pl.ds(base + i * CH, CH)], wsem)
            return 0

        lax.fori_loop(0, nrounds, body, 0, unroll=False)
        pltpu.make_async_copy(obuf, out_hbm.at[pl.ds(base, CH)], wsem).wait()

    return k(g1, g2, row3d, col3d, rad, ea, wsm)


def _sc_scatter_add(vals, row3d, zeros):
    @functools.partial(
        pl.kernel,
        out_type=jax.ShapeDtypeStruct((NC, NPAD, H), jnp.float32),
        mesh=_mesh(),
        scratch_types=(
            [pltpu.VMEM((NCH, CH), jnp.int32)]
            + [pltpu.VMEM((CH, H), jnp.float32)] * NBS
            + [pltpu.VMEM_SHARED((NPAD, H), jnp.float32)]
            + [pltpu.SemaphoreType.DMA] * NBS
        ),
    )
    def k(vals_hbm, row_hbm, zero_hbm, out_hbm, idx2d, *rest):
        bufs = rest[0:NBS]
        acc = rest[NBS]
        sems = rest[NBS + 1 : 2 * NBS + 1]
        c = lax.axis_index("c")
        s = lax.axis_index("s")
        wid = s * NC + c
        r0 = s * ROWS_PER_TILE
        base = wid * PER_W
        pltpu.sync_copy(
            zero_hbm.at[pl.ds(r0, ROWS_PER_TILE)], acc.at[pl.ds(r0, ROWS_PER_TILE)]
        )
        pltpu.sync_copy(row_hbm.at[wid], idx2d)
        for b in range(NBS):
            pltpu.async_copy(vals_hbm.at[pl.ds(base + b * CH, CH)], bufs[b], sems[b])
        plsc.subcore_barrier()

        nrounds = (NCH + NBS - 1) // NBS

        def body(j, _):
            for b in range(NBS):
                i = j * NBS + b

                @pl.when(i < NCH)
                def _():
                    pltpu.make_async_copy(
                        vals_hbm.at[pl.ds(base, CH)], bufs[b], sems[b]
                    ).wait()
                    pltpu.sync_copy(bufs[b], acc.at[idx2d.at[i]], add=True)

                    @pl.when(i + NBS < NCH)
                    def _():
                        pltpu.async_copy(
                            vals_hbm.at[pl.ds(base + (i + NBS) * CH, CH)],
                            bufs[b], sems[b],
                        )
            return 0

        lax.fori_loop(0, nrounds, body, 0, unroll=False)
        plsc.subcore_barrier()
        pltpu.sync_copy(
            acc.at[pl.ds(r0, ROWS_PER_TILE)], out_hbm.at[c, pl.ds(r0, ROWS_PER_TILE)]
        )

    return k(vals, row3d, zeros)


def _sc_scatter_equiv(phi, cdx, cdy, cdz, row3d, zeros, consts):
    @functools.partial(
        pl.kernel,
        out_type=jax.ShapeDtypeStruct((NC, NPAD, H), jnp.float32),
        mesh=_mesh(),
        scratch_types=(
            [pltpu.VMEM((NCH, CH), jnp.int32)]
            + [pltpu.VMEM((CH,), jnp.float32)] * 8
            + [pltpu.VMEM((CH, H), jnp.float32)]
            + [pltpu.VMEM((8, H), jnp.float32)]
            + [pltpu.VMEM_SHARED((NPAD, H), jnp.float32)]
            + [pltpu.SemaphoreType.DMA] * 2
        ),
    )
    def k(phi_hbm, cdx_hbm, cdy_hbm, cdz_hbm, row_hbm, zero_hbm, consts_hbm,
          out_hbm, idx2d, *rest):
        sbufs = (rest[0:4], rest[4:8])
        tbuf = rest[8]
        cv = rest[9]
        acc = rest[10]
        sems = rest[11:13]
        srcs = (phi_hbm, cdx_hbm, cdy_hbm, cdz_hbm)
        c = lax.axis_index("c")
        s = lax.axis_index("s")
        wid = s * NC + c
        r0 = s * ROWS_PER_TILE
        base = wid * PER_W
        pltpu.sync_copy(
            zero_hbm.at[pl.ds(r0, ROWS_PER_TILE)], acc.at[pl.ds(r0, ROWS_PER_TILE)]
        )
        pltpu.sync_copy(row_hbm.at[wid], idx2d)
        pltpu.sync_copy(consts_hbm, cv)
        for b in range(2):
            for q in range(4):
                pltpu.async_copy(
                    srcs[q].at[pl.ds(base + b * CH, CH)], sbufs[b][q], sems[b]
                )

        zv = cv[3, pl.ds(0, 16)]

        def zrow(r, _):
            for l in range(H // 16):
                tbuf[r, pl.ds(l * 16, 16)] = zv
            return 0

        lax.fori_loop(0, CH, zrow, 0, unroll=2)
        plsc.subcore_barrier()

        e0 = cv[0, pl.ds(0, 16)]
        e1 = cv[1, pl.ds(0, 16)]
        e2 = cv[2, pl.ds(0, 16)]

        def body(j, _):
            for b in range(2):
                i = j * 2 + b
                for q in range(4):
                    pltpu.make_async_copy(
                        srcs[q].at[pl.ds(base, CH)], sbufs[b][q], sems[b]
                    ).wait()

                def tgrp(g, _):
                    sl = pl.ds(g * 16, 16)
                    p16 = sbufs[b][0][sl]
                    sx16 = sbufs[b][1][sl]
                    sy16 = sbufs[b][2][sl]
                    sz16 = sbufs[b][3][sl]
                    for rr in range(16):
                        cd = sx16[rr] * e0 + sy16[rr] * e1 + sz16[rr] * e2
                        tbuf[g * 16 + rr, pl.ds(0, 16)] = cd * p16[rr]
                    return 0

                lax.fori_loop(0, CH // 16, tgrp, 0, unroll=False)
                pltpu.sync_copy(tbuf, acc.at[idx2d.at[i]], add=True)

                @pl.when(i + 2 < NCH)
                def _():
                    for q in range(4):
                        pltpu.async_copy(
                            srcs[q].at[pl.ds(base + (i + 2) * CH, CH)],
                            sbufs[b][q], sems[b],
                        )
            return 0

        lax.fori_loop(0, NCH // 2, body, 0, unroll=False)
        plsc.subcore_barrier()
        pltpu.sync_copy(
            acc.at[pl.ds(r0, ROWS_PER_TILE)], out_hbm.at[c, pl.ds(r0, ROWS_PER_TILE)]
        )

    return k(phi, cdx, cdy, cdz, row3d, zeros, consts)


def _tc_project(hh, wcat):
    def body(h_ref, w_ref, g1_ref, g2_ref):
        g = jnp.dot(h_ref[...], w_ref[...], preferred_element_type=jnp.float32)
        g1_ref[...] = g[:, :H]
        g2_ref[...] = g[:, H:]

    return pl.pallas_call(
        body,
        grid=(N // BN,),
        in_specs=[
            pl.BlockSpec((BN, H), lambda i: (i, 0)),
            pl.BlockSpec((H, 2 * H), lambda i: (0, 0)),
        ],
        out_specs=[pl.BlockSpec((BN, H), lambda i: (i, 0))] * 2,
        out_shape=[jax.ShapeDtypeStruct((N, H), jnp.float32)] * 2,
    )(hh, wcat)


def _tc_edge_mlp(s, w2, smalls):
    def body(s_ref, w2_ref, sm_ref, out_ref):
        i = pl.program_id(0)
        m1 = _silu(s_ref[...] + sm_ref[0:1, :])
        m = jnp.dot(m1, w2_ref[...], preferred_element_type=jnp.float32) + sm_ref[3:4, :]
        m = _silu(m)
        att = jnp.sum(m * sm_ref[4:5, :], axis=1, keepdims=True) + sm_ref[5:6, 0:1]
        ef = m * (1.0 / (1.0 + jnp.exp(-att)))
        eid = i * BE + lax.broadcasted_iota(jnp.int32, (BE, 1), 0)
        out_ref[...] = jnp.where(eid < E, ef, 0.0)

    return pl.pallas_call(
        body,
        grid=(EPAD // BE,),
        in_specs=[
            pl.BlockSpec((BE, H), lambda i: (i, 0)),
            pl.BlockSpec((H, H), lambda i: (0, 0)),
            pl.BlockSpec((8, H), lambda i: (0, 0)),
        ],
        out_specs=pl.BlockSpec((BE, H), lambda i: (i, 0)),
        out_shape=jax.ShapeDtypeStruct((EPAD, H), jnp.float32),
    )(s, w2, smalls)


def _tc_edge_equiv(s, w2, smalls):
    def body(s_ref, w2_ref, sm_ref, out_ref):
        i = pl.program_id(0)
        m1 = _silu(s_ref[...] + sm_ref[0:1, :])
        cfeat = jnp.dot(m1, w2_ref[...], preferred_element_type=jnp.float32) + sm_ref[3:4, :]
        cfeat = _silu(cfeat)
        phi_t = lax.dot_general(
            sm_ref[4:5, :], cfeat,
            dimension_numbers=(((1,), (1,)), ((), ())),
            preferred_element_type=jnp.float32,
        )
        eid = i * BE + lax.broadcasted_iota(jnp.int32, (1, BE), 1)
        out_ref[...] = jnp.where(eid < E, phi_t, 0.0)[None]

    return pl.pallas_call(
        body,
        grid=(EPAD // BE,),
        in_specs=[
            pl.BlockSpec((BE, H), lambda i: (i, 0)),
            pl.BlockSpec((H, H), lambda i: (0, 0)),
            pl.BlockSpec((8, H), lambda i: (0, 0)),
        ],
        out_specs=pl.BlockSpec((1, 1, BE), lambda i: (i, 0, 0)),
        out_shape=jax.ShapeDtypeStruct((EPAD // BE, 1, BE), jnp.float32),
    )(s, w2, smalls)


def _tc_node(hh, parts, n1, n2, smalls):
    def body(h_ref, pa_ref, pb_ref, n1_ref, n2_ref, sm_ref, out_ref):
        hcur = h_ref[...]
        agg = (pa_ref[0] + pb_ref[0]) * (1.0 / NORM_FACTOR)
        u = (
            jnp.dot(hcur, n1_ref[:H, :], preferred_element_type=jnp.float32)
            + jnp.dot(agg, n1_ref[H:, :], preferred_element_type=jnp.float32)
            + sm_ref[0:1, :]
        )
        u = _silu(u)
        u = jnp.dot(u, n2_ref[...], preferred_element_type=jnp.float32) + sm_ref[1:2, :]
        out_ref[...] = hcur + u

    return pl.pallas_call(
        body,
        grid=(N // BN,),
        in_specs=[
            pl.BlockSpec((BN, H), lambda i: (i, 0)),
            pl.BlockSpec((1, BN, H), lambda i: (0, i, 0)),
            pl.BlockSpec((1, BN, H), lambda i: (1, i, 0)),
            pl.BlockSpec((2 * H, H), lambda i: (0, 0)),
            pl.BlockSpec((H, H), lambda i: (0, 0)),
            pl.BlockSpec((8, H), lambda i: (0, 0)),
        ],
        out_specs=pl.BlockSpec((BN, H), lambda i: (i, 0)),
        out_shape=jax.ShapeDtypeStruct((N, H), jnp.float32),
    )(hh, parts, parts, n1, n2, smalls)


def _tc_final_x(xp, parts):
    def body(x_ref, pa_ref, pb_ref, out_ref):
        out_ref[...] = x_ref[...] + (pa_ref[0] + pb_ref[0]) * (1.0 / NORM_FACTOR)

    return pl.pallas_call(
        body,
        grid=(N // BN,),
        in_specs=[
            pl.BlockSpec((BN, H), lambda i: (i, 0)),
            pl.BlockSpec((1, BN, H), lambda i: (0, i, 0)),
            pl.BlockSpec((1, BN, H), lambda i: (1, i, 0)),
        ],
        out_specs=pl.BlockSpec((BN, H), lambda i: (i, 0)),
        out_shape=jax.ShapeDtypeStruct((N, H), jnp.float32),
    )(xp, parts, parts)


def _pack_gcl_smalls(p):
    z = jnp.zeros((8, H), jnp.float32)
    z = z.at[0].set(p["e1"]["b"])
    z = z.at[3].set(p["e2"]["b"])
    z = z.at[4].set(p["att"]["w"][:, 0])
    z = z.at[5].set(jnp.full((H,), p["att"]["b"][0]))
    return z


def _pack_equiv_smalls(p):
    z = jnp.zeros((8, H), jnp.float32)
    z = z.at[0].set(p["c1"]["b"])
    z = z.at[3].set(p["c2"]["b"])
    z = z.at[4].set(p["c3w"][:, 0])
    return z


def _pack_w_rows(w1):
    z = jnp.zeros((8, H), jnp.float32)
    z = z.at[0].set(w1[2 * H])
    z = z.at[1].set(w1[2 * H + 1])
    return z


def kernel(h, x, edge_index, edge_attr, params):
    row = jnp.pad(edge_index[0], (0, EPAD - E))
    col = jnp.pad(edge_index[1], (0, EPAD - E))
    ea = jnp.pad(edge_attr[:, 0], (0, EPAD - E))
    row3d = row.reshape(NW, NCH, CH)
    col3d = col.reshape(NW, NCH, CH)
    zeros_h = jnp.zeros((NPAD, H), jnp.float32)

    rad, cdx, cdy, cdz = _sc_coord(
        jnp.asarray(x[:, 0]), jnp.asarray(x[:, 1]), jnp.asarray(x[:, 2]), row, col
    )

    hh = h
    for i in range(2):
        p = params["gcl%d" % i]
        w1 = p["e1"]["w"]
        wcat = jnp.concatenate([w1[:H], w1[H : 2 * H]], axis=1)
        g1, g2 = _tc_project(hh, wcat)
        s = _sc_gather_add(g1, g2, row3d, col3d, rad, ea, _pack_w_rows(w1))
        ef = _tc_edge_mlp(s, p["e2"]["w"], _pack_gcl_smalls(p))
        parts = _sc_scatter_add(ef, row3d, zeros_h)
        hh = _tc_node(hh, parts, p["n1"]["w"], p["n2"]["w"],
                      jnp.stack([p["n1"]["b"], p["n2"]["b"]] + [jnp.zeros((H,))] * 6))

    pe = params["equiv"]
    c1 = pe["c1"]["w"]
    wcat = jnp.concatenate([c1[:H], c1[H : 2 * H]], axis=1)
    g1, g2 = _tc_project(hh, wcat)
    s = _sc_gather_add(g1, g2, row3d, col3d, rad, ea, _pack_w_rows(c1))
    phi = _tc_edge_equiv(s, pe["c2"]["w"], _pack_equiv_smalls(pe)).reshape(EPAD)
    consts = (jnp.zeros((8, H), jnp.float32)
              .at[0, 0].set(1.0).at[1, 1].set(1.0).at[2, 2].set(1.0))
    parts = _sc_scatter_equiv(phi, cdx, cdy, cdz, row3d, zeros_h, consts)
    xp = jnp.concatenate([x, jnp.zeros((N, H - 3), jnp.float32)], axis=1)
    x16 = _tc_final_x(xp, parts)
    xx = x16[:, :3]
    return hh, xx
